# baseline (speedup 1.0000x reference)
import jax
import jax.numpy as jnp
from jax import lax
from jax.experimental import pallas as pl
from jax.experimental.pallas import tpu as pltpu
from jax.experimental.pallas import tpu_sc as plsc

N = 1_000_000
NS = 16
L = 16
BV = 16
BLK = BV * L
NB = 246
CH = NB * BLK
NV = CH // L
OVL = NS * CH - N
NP = 6
PB = NB // NP
PE = PB * BLK
CAP = 512
GCAP = NS * CAP
RU = 8
MAX_NEWTON = 24
NEG_INF = float("-inf")


def _scalar(vec):
    return vec[0]


def _sc_body(x_hbm, out_hbm, chunk_v, bmax_v, blkids_v, cand_v, cnt_v,
             gcand_v, gcnt_v, gc2_v, cand_sh, cnt_sh, sems, osems):
    sid = lax.axis_index("s")
    base = jnp.minimum(sid * CH, N - CH)
    mlen = sid * CH - base
    lane0 = jnp.arange(L, dtype=jnp.int32) == 0
    neg_inf_vec = jnp.full((L,), NEG_INF, dtype=jnp.float32)

    in_copies = [
        pltpu.async_copy(x_hbm.at[pl.ds(base + p * PE, PE)],
                         chunk_v.at[pl.ds(p * PE, PE)], sems.at[p])
        for p in range(NP)]

    def bmax_body(b, mx):
        bm = chunk_v[pl.ds(b * BLK, L)]
        for j in range(1, BV):
            bm = jnp.maximum(bm, chunk_v[pl.ds(b * BLK + j * L, L)])
        bmax_v[pl.ds(b * L, L)] = bm
        return jnp.maximum(mx, bm)

    mx = neg_inf_vec
    for p in range(NP):
        in_copies[p].wait()
        if p == 0:
            def mask_body(i, _):
                chunk_v[pl.ds(i * L, L)] = neg_inf_vec
                return 0
            lax.fori_loop(0, mlen >> 4, mask_body, 0)
        with jax.named_scope("bmax"):
            mx = lax.fori_loop(p * PB, (p + 1) * PB, bmax_body, mx)

    w0 = jnp.max(mx) - jnp.float32(1.0)

    for j in range(CAP // L + 1):
        cand_v[pl.ds(j * L, L)] = neg_inf_vec

    def blkid_body(b, nact):
        bm = bmax_v[pl.ds(b * L, L)]
        hits = _scalar(plsc.all_reduce_population_count(bm >= w0))
        wm = jnp.logical_and(hits > 0, lane0)
        plsc.store_compressed(
            blkids_v.at[pl.ds(nact, L)],
            jnp.full((L,), b, dtype=jnp.int32), mask=wm)
        return nact + jnp.where(hits > 0, jnp.int32(1), jnp.int32(0))

    with jax.named_scope("blkid"):
        nact = lax.fori_loop(0, NB, blkid_body, jnp.int32(0))

    def active_body(a, cnt):
        b = blkids_v[pl.ds(a, L)][0]
        for j in range(BV):
            v = chunk_v[pl.ds(b * BLK + j * L, L)]
            m = v >= w0
            plsc.store_compressed(
                cand_v.at[pl.ds(jnp.minimum(cnt, CAP), L)], v, mask=m)
            cnt = cnt + _scalar(plsc.all_reduce_population_count(m))
        return cnt

    with jax.named_scope("active"):
        cnt = lax.fori_loop(0, nact, active_body, jnp.int32(0))
    cnt = jnp.minimum(cnt, CAP)

    with jax.named_scope("publish"):
        cnt_v[...] = jnp.full((L,), cnt, dtype=jnp.int32)
        pltpu.sync_copy(cand_v.at[pl.ds(0, CAP)], cand_sh.at[sid])
        pltpu.sync_copy(cnt_v, cnt_sh.at[sid])
        plsc.subcore_barrier()
        pltpu.sync_copy(cand_sh, gcand_v)
        pltpu.sync_copy(cnt_sh, gcnt_v)

    def list_pass(w, body_has_store, cnt0):
        def outer(state, wi):
            def inner(i, st):
                v = gcand_v[wi, pl.ds(i * L, L)]
                m = v >= w
                if body_has_store:
                    acc, c2 = st
                    plsc.store_compressed(gc2_v.at[pl.ds(c2, L)], v, mask=m)
                    c2 = c2 + _scalar(plsc.all_reduce_population_count(m))
                    return jnp.maximum(acc, v), c2
                return jnp.maximum(st, v)
            nvw = (gcnt_v[wi, :][0] + (L - 1)) >> 4
            return lax.fori_loop(0, nvw, inner, state)
        state = (neg_inf_vec, cnt0) if body_has_store else neg_inf_vec
        for wi in range(NS):
            state = outer(state, wi)
        return state

    m_glob = jnp.max(list_pass(jnp.float32(NEG_INF), False, None))
    gw0 = m_glob - jnp.float32(1.0)
    _, cnt2 = list_pass(gw0, True, jnp.int32(0))
    gc2_v[pl.ds(cnt2, L)] = neg_inf_vec

    def stats(w, cnt_in, compact):
        def body(i, st):
            s, c, c2 = st
            v = gc2_v[pl.ds(i * L, L)]
            m = v > w
            s = s + jnp.sum(jnp.where(m, v - w, jnp.float32(0.0)))
            k = _scalar(plsc.all_reduce_population_count(m))
            if compact:
                plsc.store_compressed(gc2_v.at[pl.ds(c2, L)], v, mask=m)
            return s, c + k, c2 + k
        nvi = (cnt_in + (L - 1)) >> 4
        return lax.fori_loop(0, nvi, body,
                             (jnp.float32(0.0), jnp.int32(0), jnp.int32(0)))

    def newton_update(w, s, c):
        q = jnp.full((L,), s - jnp.float32(1.0), dtype=jnp.float32) / jnp.full(
            (L,), c.astype(jnp.float32), dtype=jnp.float32)
        return w + jnp.max(q)

    s0, c0, _ = stats(gw0, cnt2, False)
    w1 = newton_update(gw0, s0, c0)

    def newton_cond(carry):
        it, w_prev, w, _ = carry
        return jnp.logical_and(it < MAX_NEWTON, w != w_prev)

    def newton_body(carry):
        it, _, w, cnt_in = carry
        s, c, cnt_new = stats(w, cnt_in, True)
        gc2_v[pl.ds(cnt_new, L)] = neg_inf_vec
        return it + 1, w, newton_update(w, s, c), cnt_new

    with jax.named_scope("newton"):
        _, _, w_fin, _ = lax.while_loop(
            newton_cond, newton_body, (jnp.int32(0), gw0, w1, cnt2))

    with jax.named_scope("restore"):
        pltpu.sync_copy(x_hbm.at[pl.ds(base, OVL)], chunk_v.at[pl.ds(0, OVL)])

    def relu_body(i, _):
        for j in range(RU):
            off = (i * RU + j) * L
            chunk_v[pl.ds(off, L)] = jnp.maximum(
                chunk_v[pl.ds(off, L)] - w_fin, jnp.float32(0.0))
        return 0

    out_copies = []
    for p in range(NP):
        with jax.named_scope("relu"):
            lax.fori_loop(p * PE // (RU * L), (p + 1) * PE // (RU * L),
                          relu_body, 0)
        out_copies.append(
            pltpu.async_copy(chunk_v.at[pl.ds(p * PE, PE)],
                             out_hbm.at[pl.ds(base + p * PE, PE)],
                             osems.at[p]))
    for c in out_copies:
        c.wait()


@jax.jit
def kernel(params):
    mesh = plsc.VectorSubcoreMesh(
        core_axis_name="c", subcore_axis_name="s", num_cores=1)
    return pl.kernel(
        _sc_body,
        out_type=jax.ShapeDtypeStruct((N,), jnp.float32),
        mesh=mesh,
        scratch_types=[
            pltpu.VMEM((CH,), jnp.float32),
            pltpu.VMEM((NB * L,), jnp.float32),
            pltpu.VMEM((NB + L,), jnp.int32),
            pltpu.VMEM((CAP + L,), jnp.float32),
            pltpu.VMEM((L,), jnp.int32),
            pltpu.VMEM((NS, CAP), jnp.float32),
            pltpu.VMEM((NS, L), jnp.int32),
            pltpu.VMEM((GCAP + L,), jnp.float32),
            pltpu.VMEM_SHARED((NS, CAP), jnp.float32),
            pltpu.VMEM_SHARED((NS, L), jnp.int32),
            pltpu.SemaphoreType.DMA((NP,)),
            pltpu.SemaphoreType.DMA((NP,)),
        ],
        compiler_params=pltpu.CompilerParams(needs_layout_passes=False),
    )(params)
